# Initial kernel scaffold; baseline (speedup 1.0000x reference)
import functools

import jax
import jax.numpy as jnp
from jax import lax
from jax.experimental import pallas as pl
from jax.experimental.pallas import tpu as pltpu
from jax.experimental.pallas import tpu_sc as plsc

NC = 2
NS = 16
NW = NC * NS
LANES = 16
CHUNK = 128


def _make_edge_kernel(HC, NH, chunks_pw, n_acc):
    KPH = HC // LANES // NH
    rows_pt = n_acc // NS
    assert rows_pt % CHUNK == 0

    mesh = plsc.VectorSubcoreMesh(core_axis_name="c", subcore_axis_name="s")

    @functools.partial(
        pl.kernel,
        out_type=(
            jax.ShapeDtypeStruct((NC, n_acc, HC), jnp.float32),
            jax.ShapeDtypeStruct((NC, n_acc, LANES), jnp.float32),
        ),
        mesh=mesh,
        scratch_types=[
            pltpu.VMEM((CHUNK, HC), jnp.float32),
            pltpu.VMEM((CHUNK, HC), jnp.float32),
            pltpu.VMEM((CHUNK, LANES), jnp.float32),
            pltpu.VMEM((NH * (HC // LANES // NH), LANES), jnp.float32),
            pltpu.VMEM((chunks_pw, CHUNK), jnp.int32),
            pltpu.VMEM((chunks_pw, CHUNK), jnp.int32),
            pltpu.VMEM_SHARED((n_acc, HC), jnp.float32),
            pltpu.VMEM_SHARED((n_acc, LANES), jnp.float32),
            pltpu.SemaphoreType.DMA,
            pltpu.SemaphoreType.DMA,
        ],
    )
    def edge_kernel(xl_hbm, xr_hbm, att_hbm, srcg, dstg, acc_out, s_out,
                    xl_buf, xr_buf, s_buf, att_buf, src_idx, dst_idx,
                    acc_sh, s_sh, sem0, sem1):
        c = lax.axis_index("c")
        sub = lax.axis_index("s")
        wid = c * NS + sub

        pltpu.sync_copy(srcg.at[wid], src_idx)
        pltpu.sync_copy(dstg.at[wid], dst_idx)
        pltpu.sync_copy(att_hbm, att_buf)

        def zero_body(i, _):
            for k in range(HC // LANES):
                xl_buf[i, pl.ds(LANES * k, LANES)] = jnp.zeros((LANES,), jnp.float32)
            s_buf[i, :] = jnp.zeros((LANES,), jnp.float32)
            return 0

        lax.fori_loop(0, CHUNK, zero_body, 0)
        base = sub * rows_pt
        for k in range(rows_pt // CHUNK):
            pltpu.sync_copy(xl_buf, acc_sh.at[pl.ds(base + k * CHUNK, CHUNK)])
            pltpu.sync_copy(s_buf, s_sh.at[pl.ds(base + k * CHUNK, CHUNK)])
        plsc.subcore_barrier()

        lane = lax.iota(jnp.int32, LANES)

        def edge_body(e, _):
            srow = jnp.zeros((LANES,), jnp.float32)
            for h in range(NH):
                ts = None
                ls = []
                for k in range(KPH):
                    col = LANES * (h * KPH + k)
                    l = xl_buf[e, pl.ds(col, LANES)]
                    r = xr_buf[e, pl.ds(col, LANES)]
                    ls.append(l)
                    z = l + r
                    lk = jnp.maximum(z, 0.2 * z)
                    t = lk * att_buf[h * KPH + k, :]
                    ts = t if ts is None else ts + t
                logit = jnp.sum(ts)
                pv = jnp.exp(jnp.full((LANES,), logit, jnp.float32))
                for k in range(KPH):
                    col = LANES * (h * KPH + k)
                    xl_buf[e, pl.ds(col, LANES)] = ls[k] * pv
                if NH == 1:
                    srow = pv
                else:
                    srow = jnp.where(lane == h, pv, srow)
            s_buf[e, :] = srow
            return 0

        def chunk_body(j, _):
            cl = pltpu.async_copy(xl_hbm.at[src_idx.at[j]], xl_buf, sem0)
            cr = pltpu.async_copy(xr_hbm.at[dst_idx.at[j]], xr_buf, sem1)
            cl.wait()
            cr.wait()
            lax.fori_loop(0, CHUNK, edge_body, 0)
            pltpu.sync_copy(xl_buf, acc_sh.at[dst_idx.at[j]], add=True)
            pltpu.sync_copy(s_buf, s_sh.at[dst_idx.at[j]], add=True)
            return 0

        lax.fori_loop(0, chunks_pw, chunk_body, 0)
        plsc.subcore_barrier()

        for k in range(rows_pt // CHUNK):
            off = base + k * CHUNK
            pltpu.sync_copy(acc_sh.at[pl.ds(off, CHUNK)],
                            acc_out.at[c, pl.ds(off, CHUNK)])
            pltpu.sync_copy(s_sh.at[pl.ds(off, CHUNK)],
                            s_out.at[c, pl.ds(off, CHUNK)])

    return edge_kernel


def _mm2_body(x_ref, wl_ref, wr_ref, xl_ref, xr_ref):
    xb = x_ref[...]
    xl_ref[...] = jnp.dot(xb, wl_ref[...], preferred_element_type=jnp.float32)
    xr_ref[...] = jnp.dot(xb, wr_ref[...], preferred_element_type=jnp.float32)


def _combine_body(a0_ref, a1_ref, s0_ref, s1_ref, emat_ref, b_ref,
                  wl_ref, wr_ref, xl_ref, xr_ref):
    acc = a0_ref[...] + a1_ref[...]
    recip = 1.0 / (s0_ref[...] + s1_ref[...] + 1e-16)
    rexp = jnp.dot(recip, emat_ref[...], preferred_element_type=jnp.float32)
    v = acc * rexp + b_ref[...]
    h = jnp.where(v > 0, v, jnp.exp(v) - 1.0)
    xl_ref[...] = jnp.dot(h, wl_ref[...], preferred_element_type=jnp.float32)
    xr_ref[...] = jnp.dot(h, wr_ref[...], preferred_element_type=jnp.float32)


def _final_body(a0_ref, a1_ref, s0_ref, s1_ref, b_ref, out_ref):
    acc = a0_ref[...] + a1_ref[...]
    recip = 1.0 / (s0_ref[...] + s1_ref[...] + 1e-16)
    v = acc * recip[:, 0:1] + b_ref[...]
    m = jnp.max(v, axis=1, keepdims=True)
    ex = jnp.exp(v - m)
    lse = jnp.log(jnp.sum(ex, axis=1, keepdims=True))
    out_ref[...] = v - m - lse


def _row_spec(bn, w):
    return pl.BlockSpec((bn, w), lambda i: (i, 0))


def _full_spec(a, b):
    return pl.BlockSpec((a, b), lambda i: (0, 0))


@jax.jit
def kernel(x, edge_inx, Wl1, Wr1, att1, b1, Wl2, Wr2, att2, b2):
    n, din = x.shape
    e = edge_inx.shape[1]
    nh1, dh1 = att1.shape
    hc1 = nh1 * dh1
    dout = att2.shape[1]

    e_tot = e + n
    chunks_pw = -(-e_tot // (CHUNK * NW))
    e_pad = chunks_pw * CHUNK * NW
    loop_idx = jnp.arange(n, dtype=edge_inx.dtype)
    pad = e_pad - e_tot
    src = jnp.concatenate([edge_inx[0], loop_idx,
                           jnp.zeros((pad,), edge_inx.dtype)])
    dst = jnp.concatenate([edge_inx[1], loop_idx,
                           jnp.full((pad,), n, edge_inx.dtype)])
    srcg = src.reshape(NW, chunks_pw, CHUNK)
    dstg = dst.reshape(NW, chunks_pw, CHUNK)

    n_acc = -(-(n + 1) // (NS * CHUNK)) * (NS * CHUNK)

    bn = 1000
    grid = (n // bn,)

    xl1, xr1 = pl.pallas_call(
        _mm2_body,
        grid=grid,
        in_specs=[_row_spec(bn, din), _full_spec(din, hc1), _full_spec(din, hc1)],
        out_specs=[_row_spec(bn, hc1), _row_spec(bn, hc1)],
        out_shape=(jax.ShapeDtypeStruct((n, hc1), jnp.float32),
                   jax.ShapeDtypeStruct((n, hc1), jnp.float32)),
    )(x, Wl1, Wr1)

    ek1 = _make_edge_kernel(hc1, nh1, chunks_pw, n_acc)
    acc1, s1 = ek1(xl1, xr1, att1, srcg, dstg)

    emat = jnp.concatenate(
        [jnp.kron(jnp.eye(nh1, dtype=jnp.float32),
                  jnp.ones((1, dh1), jnp.float32)),
         jnp.zeros((LANES - nh1, hc1), jnp.float32)], axis=0)
    xl2, xr2 = pl.pallas_call(
        _combine_body,
        grid=grid,
        in_specs=[_row_spec(bn, hc1), _row_spec(bn, hc1),
                  _row_spec(bn, LANES), _row_spec(bn, LANES),
                  _full_spec(LANES, hc1), _full_spec(1, hc1),
                  _full_spec(hc1, dout), _full_spec(hc1, dout)],
        out_specs=[_row_spec(bn, dout), _row_spec(bn, dout)],
        out_shape=(jax.ShapeDtypeStruct((n, dout), jnp.float32),
                   jax.ShapeDtypeStruct((n, dout), jnp.float32)),
    )(acc1[0, :n], acc1[1, :n], s1[0, :n], s1[1, :n], emat,
      b1.reshape(1, hc1).astype(jnp.float32), Wl2, Wr2)

    ek2 = _make_edge_kernel(dout, 1, chunks_pw, n_acc)
    acc2, s2 = ek2(xl2, xr2, att2.reshape(dout // LANES, LANES), srcg, dstg)

    out = pl.pallas_call(
        _final_body,
        grid=grid,
        in_specs=[_row_spec(bn, dout), _row_spec(bn, dout),
                  _row_spec(bn, LANES), _row_spec(bn, LANES),
                  _full_spec(1, dout)],
        out_specs=_row_spec(bn, dout),
        out_shape=jax.ShapeDtypeStruct((n, dout), jnp.float32),
    )(acc2[0, :n], acc2[1, :n], s2[0, :n], s2[1, :n],
      b2.reshape(1, dout).astype(jnp.float32))
    return out


# speedup vs baseline: 18.1380x; 18.1380x over previous
import functools

import jax
import jax.numpy as jnp
from jax import lax
from jax.experimental import pallas as pl
from jax.experimental.pallas import tpu as pltpu
from jax.experimental.pallas import tpu_sc as plsc

NC = 2
NS = 16
LANES = 16
CHUNK = 128
ROWW = 64
MSGW = ROWW + LANES


def _make_edge_kernel(NH_local, chunks_pw, n_acc, split_heads, natt):
    KPH = ROWW // LANES // NH_local
    rows_pt = n_acc // NS
    assert rows_pt % CHUNK == 0
    nworkers = NS if split_heads else NC * NS

    mesh = plsc.VectorSubcoreMesh(core_axis_name="c", subcore_axis_name="s")

    @functools.partial(
        pl.kernel,
        out_type=jax.ShapeDtypeStruct((NC, n_acc, MSGW), jnp.float32),
        mesh=mesh,
        compiler_params=pltpu.CompilerParams(needs_layout_passes=False,
                                             use_tc_tiling_on_sc=False),
        scratch_types=[
            pltpu.VMEM((CHUNK, ROWW), jnp.float32),
            pltpu.VMEM((CHUNK, ROWW), jnp.float32),
            pltpu.VMEM((CHUNK, MSGW), jnp.float32),
            pltpu.VMEM((natt, LANES), jnp.float32),
            pltpu.VMEM((chunks_pw, CHUNK), jnp.int32),
            pltpu.VMEM((chunks_pw, CHUNK), jnp.int32),
            pltpu.VMEM_SHARED((n_acc, MSGW), jnp.float32),
            pltpu.SemaphoreType.DMA,
            pltpu.SemaphoreType.DMA,
        ],
    )
    def edge_kernel(xl_hbm, xr_hbm, att_hbm, srcg, dstg, acc_out,
                    xl_buf, xr_buf, msg_buf, att_buf, src_idx, dst_idx,
                    acc_sh, sem0, sem1):
        c = lax.axis_index("c")
        sub = lax.axis_index("s")
        if split_heads:
            wid = sub
            xl_t = xl_hbm.at[c]
            xr_t = xr_hbm.at[c]
        else:
            wid = c * NS + sub
            xl_t = xl_hbm
            xr_t = xr_hbm

        pltpu.sync_copy(srcg.at[wid], src_idx)
        pltpu.sync_copy(dstg.at[wid], dst_idx)
        pltpu.sync_copy(att_hbm, att_buf)

        def zero_body(i, _):
            for k in range(MSGW // LANES):
                msg_buf[i, pl.ds(LANES * k, LANES)] = jnp.zeros((LANES,),
                                                                jnp.float32)
            return 0

        lax.fori_loop(0, CHUNK, zero_body, 0)
        base = sub * rows_pt
        for k in range(rows_pt // CHUNK):
            pltpu.sync_copy(msg_buf, acc_sh.at[pl.ds(base + k * CHUNK, CHUNK)])
        plsc.subcore_barrier()

        lane = lax.iota(jnp.int32, LANES)

        def edge_body(e, _):
            srow = jnp.zeros((LANES,), jnp.float32)
            for h in range(NH_local):
                ts = None
                ls = []
                for k in range(KPH):
                    col = LANES * (h * KPH + k)
                    l = xl_buf[e, pl.ds(col, LANES)]
                    r = xr_buf[e, pl.ds(col, LANES)]
                    ls.append(l)
                    z = l + r
                    lk = jnp.maximum(z, 0.2 * z)
                    if split_heads:
                        arow = att_buf[c * NH_local + h, :]
                    else:
                        arow = att_buf[h * KPH + k, :]
                    t = lk * arow
                    ts = t if ts is None else ts + t
                logit = jnp.sum(ts)
                pv = jnp.exp(jnp.full((LANES,), logit, jnp.float32))
                for k in range(KPH):
                    col = LANES * (h * KPH + k)
                    msg_buf[e, pl.ds(col, LANES)] = ls[k] * pv
                if NH_local == 1:
                    srow = pv
                else:
                    srow = jnp.where(lane == h, pv, srow)
            msg_buf[e, pl.ds(ROWW, LANES)] = srow
            return 0

        def chunk_body(j, _):
            cl = pltpu.async_copy(xl_t.at[src_idx.at[j]], xl_buf, sem0)
            cr = pltpu.async_copy(xr_t.at[dst_idx.at[j]], xr_buf, sem1)
            cl.wait()
            cr.wait()
            lax.fori_loop(0, CHUNK, edge_body, 0)
            pltpu.sync_copy(msg_buf, acc_sh.at[dst_idx.at[j]], add=True)
            return 0

        lax.fori_loop(0, chunks_pw, chunk_body, 0)
        plsc.subcore_barrier()

        for k in range(rows_pt // CHUNK):
            off = base + k * CHUNK
            pltpu.sync_copy(acc_sh.at[pl.ds(off, CHUNK)],
                            acc_out.at[c, pl.ds(off, CHUNK)])

    return edge_kernel


def _mm2_body(x_ref, wl_ref, wr_ref, xl_ref, xr_ref):
    xb = x_ref[...]
    xl_ref[0] = jnp.dot(xb, wl_ref[0], preferred_element_type=jnp.float32)
    xr_ref[0] = jnp.dot(xb, wr_ref[0], preferred_element_type=jnp.float32)


def _combine_body(t0_ref, t1_ref, emat_ref, b_ref, wl_ref, wr_ref,
                  xl_ref, xr_ref):
    t0 = t0_ref[...]
    t1 = t1_ref[...]
    r0 = 1.0 / (t0[:, ROWW:] + 1e-16)
    r1 = 1.0 / (t1[:, ROWW:] + 1e-16)
    em = emat_ref[...]
    v0 = t0[:, :ROWW] * jnp.dot(r0, em, preferred_element_type=jnp.float32)
    v1 = t1[:, :ROWW] * jnp.dot(r1, em, preferred_element_type=jnp.float32)
    v = jnp.concatenate([v0, v1], axis=1) + b_ref[...]
    h = jnp.where(v > 0, v, jnp.exp(v) - 1.0)
    xl_ref[...] = jnp.dot(h, wl_ref[...], preferred_element_type=jnp.float32)
    xr_ref[...] = jnp.dot(h, wr_ref[...], preferred_element_type=jnp.float32)


def _final_body(t0_ref, t1_ref, b_ref, out_ref):
    t0 = t0_ref[...]
    t1 = t1_ref[...]
    acc = t0[:, :ROWW] + t1[:, :ROWW]
    s = t0[:, ROWW:] + t1[:, ROWW:]
    recip = 1.0 / (s + 1e-16)
    v = acc * recip[:, 0:1] + b_ref[...]
    m = jnp.max(v, axis=1, keepdims=True)
    ex = jnp.exp(v - m)
    lse = jnp.log(jnp.sum(ex, axis=1, keepdims=True))
    out_ref[...] = v - m - lse


def _row_spec(bn, w):
    return pl.BlockSpec((bn, w), lambda i: (i, 0))


def _full_spec(a, b):
    return pl.BlockSpec((a, b), lambda i: (0, 0))


@jax.jit
def kernel(x, edge_inx, Wl1, Wr1, att1, b1, Wl2, Wr2, att2, b2):
    n, din = x.shape
    e = edge_inx.shape[1]
    nh1, dh1 = att1.shape
    hc1 = nh1 * dh1
    dout = att2.shape[1]
    nh_pc = nh1 // NC

    e_tot = e + n
    cpw2 = -(-e_tot // (CHUNK * NC * NS))
    e_pad = cpw2 * CHUNK * NC * NS
    cpw1 = cpw2 * NC
    loop_idx = jnp.arange(n, dtype=edge_inx.dtype)
    pad = e_pad - e_tot
    src = jnp.concatenate([edge_inx[0], loop_idx,
                           jnp.zeros((pad,), edge_inx.dtype)])
    dst = jnp.concatenate([edge_inx[1], loop_idx,
                           jnp.full((pad,), n, edge_inx.dtype)])
    srcg1 = src.reshape(NS, cpw1, CHUNK)
    dstg1 = dst.reshape(NS, cpw1, CHUNK)
    srcg2 = src.reshape(NC * NS, cpw2, CHUNK)
    dstg2 = dst.reshape(NC * NS, cpw2, CHUNK)

    n_acc = -(-(n + 1) // (NS * CHUNK)) * (NS * CHUNK)

    bn = 1000
    grid = (n // bn,)

    wl1s = Wl1.reshape(din, NC, ROWW).transpose(1, 0, 2)
    wr1s = Wr1.reshape(din, NC, ROWW).transpose(1, 0, 2)
    xl1, xr1 = pl.pallas_call(
        _mm2_body,
        grid=(n // bn, NC),
        in_specs=[pl.BlockSpec((bn, din), lambda i, j: (i, 0)),
                  pl.BlockSpec((1, din, ROWW), lambda i, j: (j, 0, 0)),
                  pl.BlockSpec((1, din, ROWW), lambda i, j: (j, 0, 0))],
        out_specs=[pl.BlockSpec((1, bn, ROWW), lambda i, j: (j, i, 0)),
                   pl.BlockSpec((1, bn, ROWW), lambda i, j: (j, i, 0))],
        out_shape=(jax.ShapeDtypeStruct((NC, n, ROWW), jnp.float32),
                   jax.ShapeDtypeStruct((NC, n, ROWW), jnp.float32)),
    )(x, wl1s, wr1s)

    ek1 = _make_edge_kernel(nh_pc, cpw1, n_acc, True, nh1)
    t1 = ek1(xl1, xr1, att1, srcg1, dstg1)

    emat = jnp.concatenate(
        [jnp.kron(jnp.eye(nh_pc, dtype=jnp.float32),
                  jnp.ones((1, dh1), jnp.float32)),
         jnp.zeros((LANES - nh_pc, ROWW), jnp.float32)], axis=0)
    xl2, xr2 = pl.pallas_call(
        _combine_body,
        grid=grid,
        in_specs=[_row_spec(bn, MSGW), _row_spec(bn, MSGW),
                  _full_spec(LANES, ROWW), _full_spec(1, hc1),
                  _full_spec(hc1, dout), _full_spec(hc1, dout)],
        out_specs=[_row_spec(bn, dout), _row_spec(bn, dout)],
        out_shape=(jax.ShapeDtypeStruct((n, dout), jnp.float32),
                   jax.ShapeDtypeStruct((n, dout), jnp.float32)),
    )(t1[0, :n], t1[1, :n], emat,
      b1.reshape(1, hc1).astype(jnp.float32), Wl2, Wr2)

    ek2 = _make_edge_kernel(1, cpw2, n_acc, False, dout // LANES)
    t2 = ek2(xl2, xr2, att2.reshape(dout // LANES, LANES), srcg2, dstg2)

    out = pl.pallas_call(
        _final_body,
        grid=grid,
        in_specs=[_row_spec(bn, MSGW), _row_spec(bn, MSGW),
                  _full_spec(1, dout)],
        out_specs=_row_spec(bn, dout),
        out_shape=jax.ShapeDtypeStruct((n, dout), jnp.float32),
    )(t2[0, :n], t2[1, :n], b2.reshape(1, dout).astype(jnp.float32))
    return out
